# Initial kernel scaffold; baseline (speedup 1.0000x reference)
import functools

import jax
import jax.numpy as jnp
from jax import lax
from jax.experimental import pallas as pl
from jax.experimental.pallas import tpu as pltpu
from jax.experimental.pallas import tpu_sc as plsc

N_NODES = 10000
N_EDGES = 160000
D_HID = 64

NC = 2
NS = 16
NW = NC * NS
EDGES_PER_W = N_EDGES // NW
CHUNK = 128
NFULL = EDGES_PER_W // CHUNK
TAIL = EDGES_PER_W - NFULL * CHUNK
ROWS_PER_SUB = N_NODES // NS

ROW_BLK = 2000
N_ROW_BLKS = N_NODES // ROW_BLK


def _segsum_sc(y, src, dst, zeros):
    mesh = plsc.VectorSubcoreMesh(core_axis_name="c", subcore_axis_name="s")

    @functools.partial(
        pl.kernel,
        mesh=mesh,
        out_type=jax.ShapeDtypeStruct((NC, N_NODES, D_HID), jnp.float32),
        scratch_types=[
            pltpu.VMEM((CHUNK,), jnp.int32),
            pltpu.VMEM((CHUNK,), jnp.int32),
            pltpu.VMEM((CHUNK, D_HID), jnp.float32),
            pltpu.VMEM((TAIL,), jnp.int32),
            pltpu.VMEM((TAIL,), jnp.int32),
            pltpu.VMEM((TAIL, D_HID), jnp.float32),
            pltpu.VMEM_SHARED((N_NODES, D_HID), jnp.float32),
            pltpu.SemaphoreType.DMA,
        ],
    )
    def seg(y_hbm, src_hbm, dst_hbm, zeros_hbm, out_hbm,
            srcv, dstv, rows, srcv_t, dstv_t, rows_t, acc, sem):
        cid = lax.axis_index("c")
        sid = lax.axis_index("s")
        wid = cid * NS + sid

        row0 = sid * ROWS_PER_SUB
        pltpu.sync_copy(zeros_hbm.at[pl.ds(row0, ROWS_PER_SUB)],
                        acc.at[pl.ds(row0, ROWS_PER_SUB)])
        plsc.subcore_barrier()

        base = wid * EDGES_PER_W

        def body(i, _):
            off = pl.multiple_of(base + i * CHUNK, 8)
            pltpu.sync_copy(src_hbm.at[pl.ds(off, CHUNK)], srcv)
            pltpu.sync_copy(dst_hbm.at[pl.ds(off, CHUNK)], dstv)
            pltpu.async_copy(y_hbm.at[srcv], rows, sem).wait()
            pltpu.sync_copy(rows, acc.at[dstv], add=True)
            return ()

        lax.fori_loop(0, NFULL, body, ())

        off = pl.multiple_of(base + NFULL * CHUNK, 8)
        pltpu.sync_copy(src_hbm.at[pl.ds(off, TAIL)], srcv_t)
        pltpu.sync_copy(dst_hbm.at[pl.ds(off, TAIL)], dstv_t)
        pltpu.async_copy(y_hbm.at[srcv_t], rows_t, sem).wait()
        pltpu.sync_copy(rows_t, acc.at[dstv_t], add=True)

        plsc.subcore_barrier()
        pltpu.sync_copy(acc.at[pl.ds(row0, ROWS_PER_SUB)],
                        out_hbm.at[cid, pl.ds(row0, ROWS_PER_SUB)])

    return seg(y, src, dst, zeros)


def _full(shape):
    return pl.BlockSpec(shape, lambda i: tuple(0 for _ in shape))


def _rows(width):
    return pl.BlockSpec((ROW_BLK, width), lambda i: (i, 0))


def _mm2_body(x_ref, wa_ref, wb_ref, b_ref, ya_ref, yb_ref):
    xb = x_ref[...]
    ya_ref[...] = jnp.dot(xb, wa_ref[...], preferred_element_type=jnp.float32)
    yb_ref[...] = (jnp.dot(xb, wb_ref[...], preferred_element_type=jnp.float32)
                   + b_ref[...])


def _stage_in(x, wa, wb, b, width_in, width_out):
    return pl.pallas_call(
        _mm2_body,
        grid=(N_ROW_BLKS,),
        in_specs=[_rows(width_in), _full(wa.shape), _full(wb.shape),
                  _full((1, width_out))],
        out_specs=[_rows(width_out), _rows(width_out)],
        out_shape=[jax.ShapeDtypeStruct((N_NODES, width_out), jnp.float32),
                   jax.ShapeDtypeStruct((N_NODES, width_out), jnp.float32)],
    )(x, wa, wb, b.reshape(1, -1))


def _combine_body(p0_ref, p1_ref, r_ref, wa_ref, wb_ref, b_ref,
                  ya_ref, yb_ref):
    h = jnp.maximum(p0_ref[...] + p1_ref[...] + r_ref[...], 0.0)
    ya_ref[...] = jnp.dot(h, wa_ref[...], preferred_element_type=jnp.float32)
    yb_ref[...] = (jnp.dot(h, wb_ref[...], preferred_element_type=jnp.float32)
                   + b_ref[...])


def _stage_mid(p, r, wa, wb, b, width_out):
    return pl.pallas_call(
        _combine_body,
        grid=(N_ROW_BLKS,),
        in_specs=[_rows(D_HID), _rows(D_HID), _rows(D_HID),
                  _full(wa.shape), _full(wb.shape), _full((1, width_out))],
        out_specs=[_rows(width_out), _rows(width_out)],
        out_shape=[jax.ShapeDtypeStruct((N_NODES, width_out), jnp.float32),
                   jax.ShapeDtypeStruct((N_NODES, width_out), jnp.float32)],
    )(p[0], p[1], r, wa, wb, b.reshape(1, -1))


def _combine_id_body(p0_ref, p1_ref, r_ref, wb_ref, b_ref, h_ref, rb_ref):
    h = jnp.maximum(p0_ref[...] + p1_ref[...] + r_ref[...], 0.0)
    h_ref[...] = h
    rb_ref[...] = (jnp.dot(h, wb_ref[...], preferred_element_type=jnp.float32)
                   + b_ref[...])


def _stage_mid_id(p, r, wb, b, width_out):
    return pl.pallas_call(
        _combine_id_body,
        grid=(N_ROW_BLKS,),
        in_specs=[_rows(D_HID), _rows(D_HID), _rows(D_HID),
                  _full(wb.shape), _full((1, width_out))],
        out_specs=[_rows(D_HID), _rows(width_out)],
        out_shape=[jax.ShapeDtypeStruct((N_NODES, D_HID), jnp.float32),
                   jax.ShapeDtypeStruct((N_NODES, width_out), jnp.float32)],
    )(p[0], p[1], r, wb, b.reshape(1, -1))


def _head_body(p0_ref, p1_ref, r3_ref, wrel3_ref, fc1w_ref, fc1b_ref,
               fc2w_ref, fc2b_ref, out_ref):
    agg = p0_ref[...] + p1_ref[...]
    z = jnp.dot(agg, wrel3_ref[...], preferred_element_type=jnp.float32)
    z = jnp.maximum(z + r3_ref[...], 0.0)
    z = jnp.dot(z, fc1w_ref[...], preferred_element_type=jnp.float32)
    z = jnp.maximum(z + fc1b_ref[...], 0.0)
    logits = (jnp.dot(z, fc2w_ref[...], preferred_element_type=jnp.float32)
              + fc2b_ref[...])
    m = jnp.max(logits, axis=-1, keepdims=True)
    lse = jnp.log(jnp.sum(jnp.exp(logits - m), axis=-1, keepdims=True)) + m
    out_ref[...] = logits - lse


def _stage_head(p, r3, wrel3, fc1w, fc1b, fc2w, fc2b):
    n_cls = fc2w.shape[1]
    return pl.pallas_call(
        _head_body,
        grid=(N_ROW_BLKS,),
        in_specs=[_rows(D_HID), _rows(D_HID), _rows(128),
                  _full(wrel3.shape), _full(fc1w.shape), _full((1, 128)),
                  _full(fc2w.shape), _full((1, n_cls))],
        out_specs=[_rows(n_cls)],
        out_shape=[jax.ShapeDtypeStruct((N_NODES, n_cls), jnp.float32)],
    )(p[0], p[1], r3, wrel3, fc1w, fc1b.reshape(1, -1),
      fc2w, fc2b.reshape(1, -1))[0]


def kernel(x, edge_index, W_rel1, W_root1, b1, W_rel2, W_root2, b2,
           W_rel3, W_root3, b3, fc1_W, fc1_b, fc2_W, fc2_b):
    src = edge_index[0].astype(jnp.int32)
    dst = edge_index[1].astype(jnp.int32)
    zeros = jnp.zeros((N_NODES, D_HID), jnp.float32)

    y1, r1 = _stage_in(x, W_rel1, W_root1, b1, 256, D_HID)
    p1 = _segsum_sc(y1, src, dst, zeros)

    y2, r2 = _stage_mid(p1, r1, W_rel2, W_root2, b2, D_HID)
    p2 = _segsum_sc(y2, src, dst, zeros)

    h2, r3 = _stage_mid_id(p2, r2, W_root3, b3, 128)
    p3 = _segsum_sc(h2, src, dst, zeros)

    return _stage_head(p3, r3, W_rel3, fc1_W, fc1_b, fc2_W, fc2_b)


# speedup vs baseline: 6.4394x; 6.4394x over previous
import functools

import jax
import jax.numpy as jnp
from jax import lax
from jax.experimental import pallas as pl
from jax.experimental.pallas import tpu as pltpu
from jax.experimental.pallas import tpu_sc as plsc

N_NODES = 10000
N_EDGES = 160000
D_HID = 64

NC = 2
NS = 16
NW = NC * NS
EDGES_PER_W = N_EDGES // NW
CHUNK = 128
NFULL = EDGES_PER_W // CHUNK
TAIL = EDGES_PER_W - NFULL * CHUNK
ROWS_PER_SUB = 624
ROWS_REM = N_NODES - NS * ROWS_PER_SUB
REM_ROW0 = NS * ROWS_PER_SUB

ROW_BLK = 2000
N_ROW_BLKS = N_NODES // ROW_BLK


def _segsum_sc(y, src, dst, zeros):
    mesh = plsc.VectorSubcoreMesh(core_axis_name="c", subcore_axis_name="s")

    @functools.partial(
        pl.kernel,
        mesh=mesh,
        compiler_params=pltpu.CompilerParams(use_tc_tiling_on_sc=False),
        out_type=jax.ShapeDtypeStruct((NC, N_NODES, D_HID), jnp.float32),
        scratch_types=[
            pltpu.VMEM((CHUNK,), jnp.int32),
            pltpu.VMEM((CHUNK,), jnp.int32),
            pltpu.VMEM((CHUNK, D_HID), jnp.float32),
            pltpu.VMEM((TAIL,), jnp.int32),
            pltpu.VMEM((TAIL,), jnp.int32),
            pltpu.VMEM((TAIL, D_HID), jnp.float32),
            pltpu.VMEM_SHARED((N_NODES, D_HID), jnp.float32),
            pltpu.SemaphoreType.DMA,
        ],
    )
    def seg(y_hbm, src_hbm, dst_hbm, zeros_hbm, out_hbm,
            srcv, dstv, rows, srcv_t, dstv_t, rows_t, acc, sem):
        cid = lax.axis_index("c")
        sid = lax.axis_index("s")
        wid = cid * NS + sid

        row0 = pl.multiple_of(sid * ROWS_PER_SUB, 8)
        pltpu.sync_copy(zeros_hbm.at[pl.ds(row0, ROWS_PER_SUB)],
                        acc.at[pl.ds(row0, ROWS_PER_SUB)])

        @pl.when(sid == NS - 1)
        def _zero_rem():
            pltpu.sync_copy(zeros_hbm.at[pl.ds(REM_ROW0, ROWS_REM)],
                            acc.at[pl.ds(REM_ROW0, ROWS_REM)])

        plsc.subcore_barrier()

        base = wid * EDGES_PER_W

        def body(i, _):
            off = pl.multiple_of(base + i * CHUNK, 8)
            pltpu.sync_copy(src_hbm.at[pl.ds(off, CHUNK)], srcv)
            pltpu.sync_copy(dst_hbm.at[pl.ds(off, CHUNK)], dstv)
            pltpu.async_copy(y_hbm.at[srcv], rows, sem).wait()
            pltpu.sync_copy(rows, acc.at[dstv], add=True)
            return ()

        lax.fori_loop(0, NFULL, body, ())

        off = pl.multiple_of(base + NFULL * CHUNK, 8)
        pltpu.sync_copy(src_hbm.at[pl.ds(off, TAIL)], srcv_t)
        pltpu.sync_copy(dst_hbm.at[pl.ds(off, TAIL)], dstv_t)
        pltpu.async_copy(y_hbm.at[srcv_t], rows_t, sem).wait()
        pltpu.sync_copy(rows_t, acc.at[dstv_t], add=True)

        plsc.subcore_barrier()
        pltpu.sync_copy(acc.at[pl.ds(row0, ROWS_PER_SUB)],
                        out_hbm.at[cid, pl.ds(row0, ROWS_PER_SUB)])

        @pl.when(sid == NS - 1)
        def _out_rem():
            pltpu.sync_copy(acc.at[pl.ds(REM_ROW0, ROWS_REM)],
                            out_hbm.at[cid, pl.ds(REM_ROW0, ROWS_REM)])

    return seg(y, src, dst, zeros)


def _full(shape):
    return pl.BlockSpec(shape, lambda i: tuple(0 for _ in shape))


def _rows(width):
    return pl.BlockSpec((ROW_BLK, width), lambda i: (i, 0))


def _mm2_body(x_ref, wa_ref, wb_ref, b_ref, ya_ref, yb_ref):
    xb = x_ref[...]
    ya_ref[...] = jnp.dot(xb, wa_ref[...], preferred_element_type=jnp.float32)
    yb_ref[...] = (jnp.dot(xb, wb_ref[...], preferred_element_type=jnp.float32)
                   + b_ref[...])


def _stage_in(x, wa, wb, b, width_in, width_out):
    return pl.pallas_call(
        _mm2_body,
        grid=(N_ROW_BLKS,),
        in_specs=[_rows(width_in), _full(wa.shape), _full(wb.shape),
                  _full((1, width_out))],
        out_specs=[_rows(width_out), _rows(width_out)],
        out_shape=[jax.ShapeDtypeStruct((N_NODES, width_out), jnp.float32),
                   jax.ShapeDtypeStruct((N_NODES, width_out), jnp.float32)],
    )(x, wa, wb, b.reshape(1, -1))


def _combine_body(p0_ref, p1_ref, r_ref, wa_ref, wb_ref, b_ref,
                  ya_ref, yb_ref):
    h = jnp.maximum(p0_ref[...] + p1_ref[...] + r_ref[...], 0.0)
    ya_ref[...] = jnp.dot(h, wa_ref[...], preferred_element_type=jnp.float32)
    yb_ref[...] = (jnp.dot(h, wb_ref[...], preferred_element_type=jnp.float32)
                   + b_ref[...])


def _stage_mid(p, r, wa, wb, b, width_out):
    return pl.pallas_call(
        _combine_body,
        grid=(N_ROW_BLKS,),
        in_specs=[_rows(D_HID), _rows(D_HID), _rows(D_HID),
                  _full(wa.shape), _full(wb.shape), _full((1, width_out))],
        out_specs=[_rows(width_out), _rows(width_out)],
        out_shape=[jax.ShapeDtypeStruct((N_NODES, width_out), jnp.float32),
                   jax.ShapeDtypeStruct((N_NODES, width_out), jnp.float32)],
    )(p[0], p[1], r, wa, wb, b.reshape(1, -1))


def _combine_id_body(p0_ref, p1_ref, r_ref, wb_ref, b_ref, h_ref, rb_ref):
    h = jnp.maximum(p0_ref[...] + p1_ref[...] + r_ref[...], 0.0)
    h_ref[...] = h
    rb_ref[...] = (jnp.dot(h, wb_ref[...], preferred_element_type=jnp.float32)
                   + b_ref[...])


def _stage_mid_id(p, r, wb, b, width_out):
    return pl.pallas_call(
        _combine_id_body,
        grid=(N_ROW_BLKS,),
        in_specs=[_rows(D_HID), _rows(D_HID), _rows(D_HID),
                  _full(wb.shape), _full((1, width_out))],
        out_specs=[_rows(D_HID), _rows(width_out)],
        out_shape=[jax.ShapeDtypeStruct((N_NODES, D_HID), jnp.float32),
                   jax.ShapeDtypeStruct((N_NODES, width_out), jnp.float32)],
    )(p[0], p[1], r, wb, b.reshape(1, -1))


def _head_body(p0_ref, p1_ref, r3_ref, wrel3_ref, fc1w_ref, fc1b_ref,
               fc2w_ref, fc2b_ref, out_ref):
    agg = p0_ref[...] + p1_ref[...]
    z = jnp.dot(agg, wrel3_ref[...], preferred_element_type=jnp.float32)
    z = jnp.maximum(z + r3_ref[...], 0.0)
    z = jnp.dot(z, fc1w_ref[...], preferred_element_type=jnp.float32)
    z = jnp.maximum(z + fc1b_ref[...], 0.0)
    logits = (jnp.dot(z, fc2w_ref[...], preferred_element_type=jnp.float32)
              + fc2b_ref[...])
    m = jnp.max(logits, axis=-1, keepdims=True)
    lse = jnp.log(jnp.sum(jnp.exp(logits - m), axis=-1, keepdims=True)) + m
    out_ref[...] = logits - lse


def _stage_head(p, r3, wrel3, fc1w, fc1b, fc2w, fc2b):
    n_cls = fc2w.shape[1]
    return pl.pallas_call(
        _head_body,
        grid=(N_ROW_BLKS,),
        in_specs=[_rows(D_HID), _rows(D_HID), _rows(128),
                  _full(wrel3.shape), _full(fc1w.shape), _full((1, 128)),
                  _full(fc2w.shape), _full((1, n_cls))],
        out_specs=[_rows(n_cls)],
        out_shape=[jax.ShapeDtypeStruct((N_NODES, n_cls), jnp.float32)],
    )(p[0], p[1], r3, wrel3, fc1w, fc1b.reshape(1, -1),
      fc2w, fc2b.reshape(1, -1))[0]


def kernel(x, edge_index, W_rel1, W_root1, b1, W_rel2, W_root2, b2,
           W_rel3, W_root3, b3, fc1_W, fc1_b, fc2_W, fc2_b):
    src = edge_index[0].astype(jnp.int32)
    dst = edge_index[1].astype(jnp.int32)
    zeros = jnp.zeros((N_NODES, D_HID), jnp.float32)

    y1, r1 = _stage_in(x, W_rel1, W_root1, b1, 256, D_HID)
    p1 = _segsum_sc(y1, src, dst, zeros)

    y2, r2 = _stage_mid(p1, r1, W_rel2, W_root2, b2, D_HID)
    p2 = _segsum_sc(y2, src, dst, zeros)

    h2, r3 = _stage_mid_id(p2, r2, W_root3, b3, 128)
    p3 = _segsum_sc(h2, src, dst, zeros)

    return _stage_head(p3, r3, W_rel3, fc1_W, fc1_b, fc2_W, fc2_b)
